# Initial kernel scaffold; baseline (speedup 1.0000x reference)
import jax
import jax.numpy as jnp
from jax.experimental import pallas as pl


def kernel(input_ids, weight):
    raise NotImplementedError("write your pallas kernel here")


# speedup vs baseline: 1.8382x; 1.8382x over previous
import functools

import jax
import jax.numpy as jnp
from jax import lax
from jax.experimental import pallas as pl
from jax.experimental.pallas import tpu as pltpu
from jax.experimental.pallas import tpu_sc as plsc

_NUM_CORES = 2
_NUM_SUBCORES = 16
_NW = _NUM_CORES * _NUM_SUBCORES

_CHUNK = 800


def _gather_rows(ids_flat, weight):
    n, d = ids_flat.shape[0], weight.shape[1]
    assert n % (_NW * 8) == 0
    b_per_w = n // _NW
    chunk = _CHUNK if b_per_w % _CHUNK == 0 else b_per_w
    n_chunks = b_per_w // chunk

    mesh = plsc.VectorSubcoreMesh(
        core_axis_name="c", subcore_axis_name="s",
        num_cores=_NUM_CORES, num_subcores=_NUM_SUBCORES)

    @functools.partial(
        pl.kernel,
        out_type=jax.ShapeDtypeStruct((n, d), jnp.float32),
        mesh=mesh,
        scratch_types=[
            pltpu.VMEM((chunk,), jnp.int32),
            pltpu.VMEM((chunk, d), jnp.float32),
            pltpu.SemaphoreType.DMA,
        ],
        compiler_params=pltpu.CompilerParams(use_tc_tiling_on_sc=False),
    )
    def k(ids_hbm, table_hbm, out_hbm, idx_v, rows_v, sem):
        wid = lax.axis_index("s") * _NUM_CORES + lax.axis_index("c")
        base = wid * b_per_w

        def body(i, carry):
            off = base + i * chunk
            pltpu.sync_copy(ids_hbm.at[pl.ds(off, chunk)], idx_v)
            pltpu.async_copy(table_hbm.at[idx_v], rows_v, sem).wait()
            pltpu.sync_copy(rows_v, out_hbm.at[pl.ds(off, chunk)])
            return carry

        lax.fori_loop(0, n_chunks, body, 0)

    return k(ids_flat, weight)


def kernel(input_ids, weight):
    b, l = input_ids.shape
    ids_flat = input_ids.reshape(b * l)
    out = _gather_rows(ids_flat, weight)
    return out.reshape(b, l, weight.shape[1])


# speedup vs baseline: 1.8691x; 1.0168x over previous
import functools

import jax
import jax.numpy as jnp
from jax import lax
from jax.experimental import pallas as pl
from jax.experimental.pallas import tpu as pltpu
from jax.experimental.pallas import tpu_sc as plsc

_NUM_CORES = 2
_NUM_SUBCORES = 16
_NW = _NUM_CORES * _NUM_SUBCORES

_CHUNK = 800


def _gather_rows(ids_flat, weight):
    n, d = ids_flat.shape[0], weight.shape[1]
    assert n % (_NW * 8) == 0
    b_per_w = n // _NW
    chunk = _CHUNK if b_per_w % _CHUNK == 0 else b_per_w
    n_chunks = b_per_w // chunk

    mesh = plsc.VectorSubcoreMesh(
        core_axis_name="c", subcore_axis_name="s",
        num_cores=_NUM_CORES, num_subcores=_NUM_SUBCORES)

    nbuf = 2
    assert n_chunks % nbuf == 0

    @functools.partial(
        pl.kernel,
        out_type=jax.ShapeDtypeStruct((n, d), jnp.float32),
        mesh=mesh,
        scratch_types=[
            [pltpu.VMEM((chunk,), jnp.int32) for _ in range(nbuf)],
            [pltpu.VMEM((chunk, d), jnp.float32) for _ in range(nbuf)],
            [pltpu.SemaphoreType.DMA for _ in range(nbuf)],
            [pltpu.SemaphoreType.DMA for _ in range(nbuf)],
        ],
        compiler_params=pltpu.CompilerParams(use_tc_tiling_on_sc=False),
    )
    def k(ids_hbm, table_hbm, out_hbm, idx_v, rows_v, gsem, ssem):
        wid = lax.axis_index("s") * _NUM_CORES + lax.axis_index("c")
        base = wid * b_per_w

        def start_gather(i, b):
            pltpu.sync_copy(ids_hbm.at[pl.ds(base + i * chunk, chunk)],
                            idx_v[b])
            pltpu.async_copy(table_hbm.at[idx_v[b]], rows_v[b], gsem[b])

        for b in range(nbuf):
            start_gather(b, b)

        def body(j, carry):
            for b in range(nbuf):
                i = j * nbuf + b
                pltpu.make_async_copy(
                    table_hbm.at[idx_v[b]], rows_v[b], gsem[b]).wait()
                pltpu.async_copy(
                    rows_v[b], out_hbm.at[pl.ds(base + i * chunk, chunk)],
                    ssem[b])

                @pl.when(i + nbuf < n_chunks)
                def _():
                    pltpu.make_async_copy(
                        rows_v[b],
                        out_hbm.at[pl.ds(base + i * chunk, chunk)],
                        ssem[b]).wait()
                    start_gather(i + nbuf, b)
            return carry

        lax.fori_loop(0, n_chunks // nbuf, body, 0)

        for b in range(nbuf):
            i = n_chunks - nbuf + b
            pltpu.make_async_copy(
                rows_v[b], out_hbm.at[pl.ds(base + i * chunk, chunk)],
                ssem[b]).wait()

    return k(ids_flat, weight)


def kernel(input_ids, weight):
    b, l = input_ids.shape
    ids_flat = input_ids.reshape(b * l)
    out = _gather_rows(ids_flat, weight)
    return out.reshape(b, l, weight.shape[1])
